# baseline (speedup 1.0000x reference)
import functools

import jax
import jax.numpy as jnp
from jax import lax
from jax.experimental import pallas as pl
from jax.experimental.pallas import tpu as pltpu
from jax.experimental.pallas import tpu_sc as plsc

NUM_CORES = 2
NUM_SUBCORES = 16
LANES = 16
NW = NUM_CORES * NUM_SUBCORES

BATCH = 16384
EMBED = 64
B_PER_W = BATCH // NW
CHUNK = 128
N_CHUNKS = B_PER_W // CHUNK
N_GROUPS = B_PER_W // LANES


def _mf_body(u_idx_hbm, i_idx_hbm, user_emb, user_bias, item_emb, item_bias,
             mean_hbm, out_hbm,
             u_idx_v, i_idx_v, u_rows, i_rows, bu_v, bi_v, mean_v, out_v, sem):
    wid = lax.axis_index("s") * NUM_CORES + lax.axis_index("c")

    pltpu.sync_copy(u_idx_hbm.at[wid], u_idx_v)
    pltpu.sync_copy(i_idx_hbm.at[wid], i_idx_v)
    pltpu.sync_copy(mean_hbm, mean_v)

    copies = []
    for j in range(N_CHUNKS):
        rows = pl.ds(j * CHUNK, CHUNK)
        copies.append(pltpu.async_copy(user_emb.at[u_idx_v.at[j]], u_rows.at[rows], sem))
        copies.append(pltpu.async_copy(item_emb.at[i_idx_v.at[j]], i_rows.at[rows], sem))
        copies.append(pltpu.async_copy(user_bias.at[u_idx_v.at[j]], bu_v.at[rows], sem))
        copies.append(pltpu.async_copy(item_bias.at[i_idx_v.at[j]], bi_v.at[rows], sem))
    for c in copies:
        c.wait()

    mean_vec = mean_v[...]

    def group(g, carry):
        row = g * LANES + lax.iota(jnp.int32, LANES)
        acc = (bu_v[pl.ds(g * LANES, LANES)]
               + bi_v[pl.ds(g * LANES, LANES)]
               + mean_vec)
        for d in range(EMBED):
            col = jnp.full((LANES,), d, jnp.int32)
            acc = acc + (plsc.load_gather(u_rows, [row, col])
                         * plsc.load_gather(i_rows, [row, col]))
        out_v[pl.ds(g * LANES, LANES)] = acc
        return carry

    lax.fori_loop(0, N_GROUPS, group, 0)

    pltpu.sync_copy(out_v, out_hbm.at[wid])


@jax.jit
def _mf_sc(u_idx, i_idx, user_emb, user_bias, item_emb, item_bias, mean):
    mesh = plsc.VectorSubcoreMesh(core_axis_name="c", subcore_axis_name="s",
                                  num_cores=NUM_CORES, num_subcores=NUM_SUBCORES)
    run = functools.partial(
        pl.kernel,
        out_type=jax.ShapeDtypeStruct((NW, B_PER_W), jnp.float32),
        mesh=mesh,
        scratch_types=[
            pltpu.VMEM((N_CHUNKS, CHUNK), jnp.int32),
            pltpu.VMEM((N_CHUNKS, CHUNK), jnp.int32),
            pltpu.VMEM((B_PER_W, EMBED), jnp.float32),
            pltpu.VMEM((B_PER_W, EMBED), jnp.float32),
            pltpu.VMEM((B_PER_W,), jnp.float32),
            pltpu.VMEM((B_PER_W,), jnp.float32),
            pltpu.VMEM((LANES,), jnp.float32),
            pltpu.VMEM((B_PER_W,), jnp.float32),
            pltpu.SemaphoreType.DMA,
        ],
        compiler_params=pltpu.CompilerParams(needs_layout_passes=False,
                                             use_tc_tiling_on_sc=False),
    )(_mf_body)
    return run(u_idx, i_idx, user_emb, user_bias, item_emb, item_bias, mean)


def kernel(u_id, i_id, user_emb, user_bias, item_emb, item_bias, mean):
    u_idx = u_id.astype(jnp.int32).reshape(NW, N_CHUNKS, CHUNK)
    i_idx = i_id.astype(jnp.int32).reshape(NW, N_CHUNKS, CHUNK)
    mean16 = jnp.broadcast_to(mean.astype(jnp.float32).reshape(1), (LANES,))
    out = _mf_sc(u_idx, i_idx, user_emb, user_bias.reshape(-1),
                 item_emb, item_bias.reshape(-1), mean16)
    return out.reshape(BATCH)
